# baseline (speedup 1.0000x reference)
import functools
import math

import jax
import jax.numpy as jnp
from jax import lax
from jax.experimental import pallas as pl
from jax.experimental.pallas import tpu as pltpu
from jax.experimental.pallas import tpu_sc as plsc

NC = 2
NS = 16
LANES = 16

CHUNK = 16
NBUF = 7
WDEFER = 2


@functools.partial(jax.jit, static_argnames=("b", "s", "d"))
def _emb_lookup(x, table, b, s, d):
    n_rows = b * s
    nw = NC * NS
    per_w = n_rows // nw
    w_per_row = s // per_w
    n_chunks = per_w // CHUNK
    nbuf = min(NBUF, n_chunks)
    scale = float(math.sqrt(d))
    mesh = plsc.VectorSubcoreMesh(
        core_axis_name="c", subcore_axis_name="s",
        num_cores=NC, num_subcores=NS)

    @functools.partial(
        pl.kernel,
        out_type=jax.ShapeDtypeStruct((b, s, d), jnp.float32),
        mesh=mesh,
        scratch_types=(
            [pltpu.VMEM((per_w,), jnp.int32)]
            + [pltpu.VMEM((CHUNK, d), jnp.float32) for _ in range(nbuf)]
            + [pltpu.SemaphoreType.DMA for _ in range(2 * nbuf)]
        ),
    )
    def k(idx_hbm, table_hbm, out_hbm, idx_v, *rest):
        bufs = rest[:nbuf]
        gsems = rest[nbuf:2 * nbuf]
        wsems = rest[2 * nbuf:]
        wid = lax.axis_index("s") * NC + lax.axis_index("c")
        base = wid * per_w
        row = wid // w_per_row
        col = (wid % w_per_row) * per_w
        pltpu.sync_copy(idx_hbm.at[row, pl.ds(col, per_w)], idx_v)

        def issue_gather(g, i):
            return pltpu.async_copy(
                table_hbm.at[idx_v.at[pl.ds(g * CHUNK, CHUNK)]],
                bufs[i], gsems[i])

        def scale_buf(buf):
            def scale_rows(r2, carry):
                for j in range(d // LANES):
                    sl = pl.ds(j * LANES, LANES)
                    buf[2 * r2, sl] = buf[2 * r2, sl] * scale
                    buf[2 * r2 + 1, sl] = buf[2 * r2 + 1, sl] * scale
                return carry
            lax.fori_loop(0, CHUNK // 2, scale_rows, 0)

        ghandles = {}
        whandles = {}
        for g in range(nbuf):
            ghandles[g] = issue_gather(g, g)
        for g in range(n_chunks):
            i = g % nbuf
            ghandles[g].wait()
            prev = g - WDEFER
            if prev >= 0 and prev + nbuf < n_chunks:
                whandles[prev].wait()
                ghandles[prev + nbuf] = issue_gather(prev + nbuf, prev % nbuf)
            scale_buf(bufs[i])
            whandles[g] = pltpu.async_copy(
                bufs[i],
                out_hbm.at[row, pl.ds(col + g * CHUNK, CHUNK)], wsems[i])
        for g in range(max(0, n_chunks - nbuf), n_chunks):
            whandles[g].wait()

    return k(x, table)


def kernel(x, table):
    b, s = x.shape
    v, d = table.shape
    if x.dtype != jnp.int32:
        x = x.astype(jnp.int32)
    return _emb_lookup(x, table, b=b, s=s, d=d)


# speedup vs baseline: 1.3490x; 1.3490x over previous
import functools
import math

import jax
import jax.numpy as jnp
from jax import lax
from jax.experimental import pallas as pl
from jax.experimental.pallas import tpu as pltpu
from jax.experimental.pallas import tpu_sc as plsc

NC = 2
NS = 16
LANES = 16

CHUNK = 16
NBUF = 7
WDEFER = 2


@functools.partial(jax.jit, static_argnames=("b", "s", "d"))
def _emb_lookup(x, table, b, s, d):
    n_rows = b * s
    nw = NC * NS
    per_w = n_rows // nw
    w_per_row = s // per_w
    n_chunks = per_w // CHUNK
    nbuf = min(NBUF, n_chunks)
    scale = float(math.sqrt(d))
    mesh = plsc.VectorSubcoreMesh(
        core_axis_name="c", subcore_axis_name="s",
        num_cores=NC, num_subcores=NS)

    @functools.partial(
        pl.kernel,
        out_type=jax.ShapeDtypeStruct((b, s, d), jnp.float32),
        mesh=mesh,
        scratch_types=(
            pltpu.VMEM((per_w,), jnp.int32),
            pltpu.VMEM((nbuf * CHUNK, d), jnp.float32),
            pltpu.SemaphoreType.DMA((nbuf,)),
            pltpu.SemaphoreType.DMA((nbuf,)),
        ),
    )
    def k(idx_hbm, table_hbm, out_hbm, idx_v, bufall, gsem, wsem):
        wid = lax.axis_index("s") * NC + lax.axis_index("c")
        row = wid // w_per_row
        col = (wid % w_per_row) * per_w
        pltpu.sync_copy(idx_hbm.at[row, pl.ds(col, per_w)], idx_v)

        def gather_copy(g, slot):
            return pltpu.make_async_copy(
                table_hbm.at[idx_v.at[pl.ds(g * CHUNK, CHUNK)]],
                bufall.at[pl.ds(slot * CHUNK, CHUNK)], gsem.at[slot])

        def wb_copy(g, slot):
            return pltpu.make_async_copy(
                bufall.at[pl.ds(slot * CHUNK, CHUNK)],
                out_hbm.at[row, pl.ds(col + g * CHUNK, CHUNK)], wsem.at[slot])

        def prime(g, carry):
            gather_copy(g, g).start()
            return carry
        lax.fori_loop(0, nbuf, prime, 0)

        def step(g, carry):
            slot = lax.rem(g, nbuf)
            gather_copy(g, slot).wait()
            prev = g - WDEFER
            pslot = lax.rem(prev + nbuf, nbuf)

            @pl.when((prev >= 0) & (prev + nbuf < n_chunks))
            def _():
                wb_copy(prev, pslot).wait()
                gather_copy(prev + nbuf, pslot).start()

            rbase = slot * CHUNK

            def scale_row(r, c2):
                for j in range(d // LANES):
                    sl = pl.ds(j * LANES, LANES)
                    bufall[rbase + r, sl] = bufall[rbase + r, sl] * scale
                return c2
            lax.fori_loop(0, CHUNK, scale_row, 0)

            wb_copy(g, slot).start()
            return carry
        lax.fori_loop(0, n_chunks, step, 0)

        def drain(g, carry):
            wb_copy(g, lax.rem(g, nbuf)).wait()
            return carry
        lax.fori_loop(max(0, n_chunks - nbuf), n_chunks, drain, 0)

    return k(x, table)


def kernel(x, table):
    b, s = x.shape
    v, d = table.shape
    if x.dtype != jnp.int32:
        x = x.astype(jnp.int32)
    return _emb_lookup(x, table, b=b, s=s, d=d)
